# Initial kernel scaffold; baseline (speedup 1.0000x reference)
import jax
import jax.numpy as jnp
from jax.experimental import pallas as pl


def kernel(cosine, label, qs_scores):
    raise NotImplementedError("write your pallas kernel here")


# speedup vs baseline: 5.5075x; 5.5075x over previous
import functools

import jax
import jax.numpy as jnp
from jax.experimental import pallas as pl

_S = 64.0
_M = 0.4

_RB = 256
_CB = 2048


def _scale_margin_kernel(lab_ref, neg_ref, cos_ref, out_ref):
    j = pl.program_id(1)
    col0 = j * _CB
    cols = col0 + jax.lax.broadcasted_iota(jnp.int32, (_RB, _CB), 1)
    hit = cols == lab_ref[...]
    out_ref[...] = cos_ref[...] * _S + jnp.where(hit, neg_ref[...], 0.0)


def kernel(cosine, label, qs_scores):
    del qs_scores
    B, C = cosine.shape
    mkey = jax.random.fold_in(jax.random.key(0), 123)
    margin = _M + 0.05 * jax.random.normal(mkey, (B, 1), dtype=jnp.float32)
    neg = -_S * margin
    lab2 = label.reshape(B, 1)

    grid = (B // _RB, pl.cdiv(C, _CB))
    return pl.pallas_call(
        _scale_margin_kernel,
        grid=grid,
        in_specs=[
            pl.BlockSpec((_RB, 1), lambda i, j: (i, 0)),
            pl.BlockSpec((_RB, 1), lambda i, j: (i, 0)),
            pl.BlockSpec((_RB, _CB), lambda i, j: (i, j)),
        ],
        out_specs=pl.BlockSpec((_RB, _CB), lambda i, j: (i, j)),
        out_shape=jax.ShapeDtypeStruct((B, C), cosine.dtype),
    )(lab2, neg, cosine)
